# baseline (speedup 1.0000x reference)
import functools

import jax
import jax.numpy as jnp
from jax import lax
from jax.experimental import pallas as pl
from jax.experimental.pallas import tpu as pltpu
from jax.experimental.pallas import tpu_sc as plsc

_BETA = 0.25


def _dist_argmin_body(zt_ref, e_ref, idx_ref, counts_ref, e2_ref):
    i = pl.program_id(0)
    zt = zt_ref[...]
    e = e_ref[...]
    r, k = zt.shape[1], e.shape[0]

    @pl.when(i == 0)
    def _():
        e2_ref[...] = jnp.sum(e * e, axis=1, keepdims=True)
        counts_ref[...] = jnp.zeros_like(counts_ref)

    x2 = jnp.sum(zt * zt, axis=0, keepdims=True)
    dot = lax.dot_general(
        e,
        zt.astype(jnp.bfloat16),
        (((1,), (0,)), ((), ())),
        precision=lax.Precision.DEFAULT,
        preferred_element_type=jnp.float32,
    )
    d2 = x2 + e2_ref[...] - 2.0 * dot
    dist = jnp.sqrt(jnp.maximum(d2, 0.0))
    dmin = jnp.min(dist, axis=0, keepdims=True)
    row = lax.broadcasted_iota(jnp.int32, (k, r), 0).astype(jnp.float32)
    cand = jnp.where(dist == dmin, row, float(k))
    idxf = jnp.min(cand, axis=0, keepdims=True)
    idx_ref[...] = idxf.astype(jnp.int32).reshape(1, 1, r)
    onehot = jnp.where(row == idxf, 1.0, 0.0)
    counts_ref[...] += jnp.sum(onehot, axis=1, keepdims=True)


def _dist_argmin(z_t, embed_weight, row_tile):
    d, n = z_t.shape
    k = embed_weight.shape[0]
    grid = n // row_tile
    return pl.pallas_call(
        _dist_argmin_body,
        grid=(grid,),
        in_specs=[
            pl.BlockSpec((d, row_tile), lambda i: (0, i)),
            pl.BlockSpec((k, d), lambda i: (0, 0)),
        ],
        out_specs=[
            pl.BlockSpec((1, 1, row_tile), lambda i: (i, 0, 0)),
            pl.BlockSpec((k, 1), lambda i: (0, 0)),
        ],
        out_shape=[
            jax.ShapeDtypeStruct((grid, 1, row_tile), jnp.int32),
            jax.ShapeDtypeStruct((k, 1), jnp.float32),
        ],
        scratch_shapes=[pltpu.VMEM((k, 1), jnp.float32)],
    )(z_t, embed_weight)


def _sc_gather(table, idx):
    n = idx.shape[0]
    k, d = table.shape
    info = plsc.get_sparse_core_info()
    nc, ns = info.num_cores, info.num_subcores
    nw = nc * ns
    b_per_w = n // nw
    mesh = plsc.VectorSubcoreMesh(core_axis_name="c", subcore_axis_name="s")

    @functools.partial(
        pl.kernel,
        mesh=mesh,
        compiler_params=pltpu.CompilerParams(use_tc_tiling_on_sc=False),
        out_type=jax.ShapeDtypeStruct((n, d), jnp.float32),
        scratch_types=[
            pltpu.VMEM((b_per_w,), jnp.int32),
            pltpu.VMEM((b_per_w, d), jnp.float32),
            pltpu.SemaphoreType.DMA,
        ],
    )
    def gather_kernel(table_hbm, idx_hbm, out_hbm, idx_v, rows_v, sem):
        wid = lax.axis_index("s") * nc + lax.axis_index("c")
        base = wid * b_per_w
        pltpu.sync_copy(idx_hbm.at[pl.ds(base, b_per_w)], idx_v)
        pltpu.async_copy(table_hbm.at[idx_v], rows_v, sem).wait()
        pltpu.sync_copy(rows_v, out_hbm.at[pl.ds(base, b_per_w)])

    return gather_kernel(table, idx)


def _finalize_body(z_ref, zq_ref, c_ref, st_ref, loss_ref):
    z = z_ref[...]
    zq = zq_ref[...]
    diff = zq - z
    st_ref[...] = z + diff
    n_elems = z.shape[0] * z.shape[1]
    mse = jnp.sum(diff * diff) / float(n_elems)
    p = c_ref[...] * (1.0 / float(z.shape[0]))
    ent = jnp.sum(p * jnp.log(p + 1e-10))
    perplexity = jnp.exp(-ent)
    loss = (_BETA * mse + mse) - 0.01 * perplexity
    loss_ref[...] = jnp.reshape(loss, (1, 1))


def _finalize(z_flat, zq, counts):
    n, d = z_flat.shape
    return pl.pallas_call(
        _finalize_body,
        out_shape=[
            jax.ShapeDtypeStruct((n, d), jnp.float32),
            jax.ShapeDtypeStruct((1, 1), jnp.float32),
        ],
    )(z_flat, zq, counts)


def kernel(z_e, embed_weight):
    b, l, d = z_e.shape
    z_flat = z_e.reshape(b * l, d)
    idx3d, counts = _dist_argmin(z_flat.T, embed_weight, row_tile=256)
    idx = idx3d.reshape(b * l)
    counts = counts.reshape(1, -1)
    zq = _sc_gather(embed_weight, idx)
    zq_st, loss = _finalize(z_flat, zq, counts)
    return (
        zq_st.reshape(b, l, d),
        loss.reshape(()),
        idx.reshape(b, l),
    )


# speedup vs baseline: 1.4079x; 1.4079x over previous
import functools

import jax
import jax.numpy as jnp
from jax import lax
from jax.experimental import pallas as pl
from jax.experimental.pallas import tpu as pltpu
from jax.experimental.pallas import tpu_sc as plsc

_BETA = 0.25


def _dist_argmin_body(zt_ref, e_ref, idx_ref, counts_ref, e2_ref):
    i = pl.program_id(0)
    zt = zt_ref[...]
    e = e_ref[...]
    r, k = zt.shape[1], e.shape[0]

    @pl.when(i == 0)
    def _():
        e2_ref[...] = jnp.sum(e * e, axis=1, keepdims=True)
        counts_ref[...] = jnp.zeros_like(counts_ref)

    dot = lax.dot_general(
        e,
        zt.astype(jnp.bfloat16),
        (((1,), (0,)), ((), ())),
        precision=lax.Precision.DEFAULT,
        preferred_element_type=jnp.float32,
    )
    score = e2_ref[...] - 2.0 * dot
    smin = jnp.min(score, axis=0, keepdims=True)
    row = lax.broadcasted_iota(jnp.int32, (k, r), 0).astype(jnp.float32)
    cand = jnp.where(score == smin, row, float(k))
    idxf = jnp.min(cand, axis=0, keepdims=True)
    idx_ref[...] = idxf.astype(jnp.int32).reshape(1, 1, r)
    onehot = jnp.where(row == idxf, 1.0, 0.0)
    counts_ref[...] += jnp.sum(onehot, axis=1, keepdims=True)


def _dist_argmin(z_t, embed_weight, row_tile):
    d, n = z_t.shape
    k = embed_weight.shape[0]
    grid = n // row_tile
    return pl.pallas_call(
        _dist_argmin_body,
        grid=(grid,),
        in_specs=[
            pl.BlockSpec((d, row_tile), lambda i: (0, i)),
            pl.BlockSpec((k, d), lambda i: (0, 0)),
        ],
        out_specs=[
            pl.BlockSpec((1, 1, row_tile), lambda i: (i, 0, 0)),
            pl.BlockSpec((k, 1), lambda i: (0, 0)),
        ],
        out_shape=[
            jax.ShapeDtypeStruct((grid, 1, row_tile), jnp.int32),
            jax.ShapeDtypeStruct((k, 1), jnp.float32),
        ],
        scratch_shapes=[pltpu.VMEM((k, 1), jnp.float32)],
    )(z_t, embed_weight)


def _sc_gather(table, idx):
    n = idx.shape[0]
    k, d = table.shape
    info = plsc.get_sparse_core_info()
    nc, ns = info.num_cores, info.num_subcores
    nw = nc * ns
    b_per_w = n // nw
    mesh = plsc.VectorSubcoreMesh(core_axis_name="c", subcore_axis_name="s")

    @functools.partial(
        pl.kernel,
        mesh=mesh,
        compiler_params=pltpu.CompilerParams(use_tc_tiling_on_sc=False),
        out_type=jax.ShapeDtypeStruct((n, d), jnp.float32),
        scratch_types=[
            pltpu.VMEM((b_per_w,), jnp.int32),
            pltpu.VMEM((b_per_w, d), jnp.float32),
            pltpu.SemaphoreType.DMA,
        ],
    )
    def gather_kernel(table_hbm, idx_hbm, out_hbm, idx_v, rows_v, sem):
        wid = lax.axis_index("s") * nc + lax.axis_index("c")
        base = wid * b_per_w
        pltpu.sync_copy(idx_hbm.at[pl.ds(base, b_per_w)], idx_v)
        pltpu.async_copy(table_hbm.at[idx_v], rows_v, sem).wait()
        pltpu.sync_copy(rows_v, out_hbm.at[pl.ds(base, b_per_w)])

    return gather_kernel(table, idx)


def _finalize_body(z_ref, zq_ref, c_ref, st_ref, loss_ref):
    z = z_ref[...]
    zq = zq_ref[...]
    diff = zq - z
    st_ref[...] = z + diff
    n_elems = z.shape[0] * z.shape[1]
    mse = jnp.sum(diff * diff) / float(n_elems)
    p = c_ref[...] * (1.0 / float(z.shape[0]))
    ent = jnp.sum(p * jnp.log(p + 1e-10))
    perplexity = jnp.exp(-ent)
    loss = (_BETA * mse + mse) - 0.01 * perplexity
    loss_ref[...] = jnp.reshape(loss, (1, 1))


def _finalize(z_flat, zq, counts):
    n, d = z_flat.shape
    return pl.pallas_call(
        _finalize_body,
        out_shape=[
            jax.ShapeDtypeStruct((n, d), jnp.float32),
            jax.ShapeDtypeStruct((1, 1), jnp.float32),
        ],
    )(z_flat, zq, counts)


def kernel(z_e, embed_weight):
    b, l, d = z_e.shape
    z_flat = z_e.reshape(b * l, d)
    idx3d, counts = _dist_argmin(z_flat.T, embed_weight, row_tile=256)
    idx = idx3d.reshape(b * l)
    counts = counts.reshape(1, -1)
    zq = _sc_gather(embed_weight, idx)
    zq_st, loss = _finalize(z_flat, zq, counts)
    return (
        zq_st.reshape(b, l, d),
        loss.reshape(()),
        idx.reshape(b, l),
    )
